# baseline (speedup 1.0000x reference)
import jax
import jax.numpy as jnp
from jax import lax
from jax.experimental import pallas as pl
from jax.experimental.pallas import tpu as pltpu
from jax.experimental.pallas import tpu_sc as plsc

_N = 16384
_K = 4915
_NW = 16
_CH = _N // _NW
_NV = _CH // 16

_ROUNDS = [(25, 7), (20, 5), (15, 5), (10, 5), (5, 5), (0, 5)]

_CP = pltpu.CompilerParams(needs_layout_passes=False)

_SPIN_LIM = jnp.int32(1 << 26)

_LOG1P = [5.237010936021136e-09, 0.9999989107138262, -0.4999622466391895,
          0.3328184407594258, -0.24635666572645035, 0.18468861839607467,
          -0.12526679277946434, 0.06651261739862285, -0.023038336454007536,
          0.0037526334774422208]


def _iota16():
    return lax.broadcasted_iota(jnp.int32, (16,), 0)


def _softplus16(z):
    w = jnp.exp(-jnp.abs(z))
    p = jnp.full(z.shape, jnp.float32(_LOG1P[-1]))
    for c in reversed(_LOG1P[:-1]):
        p = p * w + jnp.float32(c)
    return jnp.maximum(z, 0.0) + p


def _sc_body(x_hbm, t_hbm, nonce_hbm, out_hbm,
             xv, tv, kv, cbuf, hist, stage, gath0, gath, nv_vm,
             sh0, sh, shf, gathf):
    wid = lax.axis_index("s")
    base = wid * _CH

    pltpu.sync_copy(x_hbm.at[pl.ds(base, _CH)], xv)
    pltpu.sync_copy(t_hbm.at[pl.ds(base, _CH)], tv)
    pltpu.sync_copy(nonce_hbm, nv_vm)

    nonce = nv_vm[...]
    n17 = nonce & jnp.int32(0x1FFFF)
    nsh = n17 << jnp.int32(15)
    iota = _iota16()

    def prep(i, _):
        x = xv[pl.ds(i * 16, 16)]
        t = tv[pl.ds(i * 16, 16)].astype(jnp.float32)
        z = x * (1.0 - 2.0 * t)
        u = plsc.bitcast(z, jnp.uint32)
        neg = (u >> jnp.uint32(31)) == jnp.uint32(1)
        bk = jnp.where(neg, ~u, u | jnp.uint32(0x80000000))
        kv[pl.ds(i * 16, 16)] = bk
        return 0

    lax.fori_loop(0, _NV, prep, 0, unroll=8)

    P = jnp.uint32(0)
    c_above = jnp.int32(0)
    m_local = jnp.int32(_CH)
    compacted = False

    for r, (shift, width) in enumerate(_ROUNDS):
        nb = 1 << width
        nbv = nb // 16
        zero = jnp.zeros((16,), jnp.int32)

        def zbody(i, _):
            hist[pl.ds(i * 16, 16)] = zero
            return 0

        lax.fori_loop(0, nb, zbody, 0, unroll=8)

        ones = jnp.ones((16,), jnp.int32)
        lane_off = iota * nb
        mask_u = jnp.uint32((1 << width) - 1)

        if not compacted:
            def hbody(i, _):
                bk = kv[pl.ds(i * 16, 16)]
                b = ((bk >> jnp.uint32(shift)) & mask_u).astype(jnp.int32)
                plsc.addupdate_scatter(hist, [lane_off + b], ones)
                return 0

            lax.fori_loop(0, _NV, hbody, 0, unroll=8)
        else:
            pmask = jnp.uint32((~((1 << (shift + width)) - 1)) & 0xFFFFFFFF)
            Pcur = P

            def hbody(i, _):
                bk = cbuf[pl.ds(i * 16, 16)]
                inb = ((i * 16 + iota) < m_local) & ((bk & pmask) == Pcur)
                b = ((bk >> jnp.uint32(shift)) & mask_u).astype(jnp.int32)
                plsc.addupdate_scatter(hist, [lane_off + b], ones, mask=inb)
                return 0

            nvc = (m_local + jnp.int32(15)) >> jnp.int32(4)
            lax.fori_loop(0, nvc, hbody, 0, unroll=False)

        for j in range(nbv):
            acc = jnp.zeros((16,), jnp.int32)
            for l in range(16):
                acc = acc + hist[pl.ds(l * nb + j * 16, 16)]
            stage[pl.ds(j * 16, 16)] = acc | nsh

        if r == 0:
            pltpu.sync_copy(stage.at[pl.ds(0, nb)], sh0.at[pl.ds(wid * 128, 128)])
        else:
            pltpu.sync_copy(stage.at[pl.ds(0, nb)],
                            sh.at[pl.ds(((r - 1) * 16 + wid) * 32, 32)])

        g = gath0 if r == 0 else gath

        def read_and_check(_unused):
            if r == 0:
                pltpu.sync_copy(sh0, g)
            else:
                pltpu.sync_copy(sh.at[pl.ds((r - 1) * 512, 512)], g)
            okacc = jnp.zeros((16,), jnp.int32)
            for row in range(16):
                for j in range(nbv):
                    w = g[pl.ds(row * nb + j * 16, 16)]
                    okacc = okacc + jnp.where(
                        lax.shift_right_logical(w, jnp.int32(15)) == n17, 1, 0)
            return jnp.sum(okacc)

        okc = read_and_check(0)
        okc, _sp = lax.while_loop(
            lambda cs: (cs[0] != jnp.int32(16 * nb)) & (cs[1] < _SPIN_LIM),
            lambda cs: (read_and_check(0), cs[1] + 1),
            (okc, jnp.int32(0)))

        hv = []
        for j in range(nbv):
            acc = jnp.zeros((16,), jnp.int32)
            for row in range(16):
                acc = acc + (g[pl.ds(row * nb + j * 16, 16)] & jnp.int32(0x7FFF))
            hv.append(acc)

        need = jnp.int32(_K) - c_above
        rt = jnp.int32(0)
        bstar = jnp.int32(0)
        above_sel = jnp.int32(0)
        for j in reversed(range(nbv)):
            h = hv[j]
            rv = lax.rev(h, (0,))
            cs = plsc.cumsum(rv)
            a = lax.rev(cs, (0,)) + rt
            bmask = (a >= need) & ((a - h) < need)
            found = jnp.sum(jnp.where(bmask, 1, 0))
            bidx = jnp.sum(jnp.where(bmask, iota, 0))
            abv = jnp.sum(jnp.where(bmask, a - h, 0))
            hit = found > 0
            bstar = jnp.where(hit, j * 16 + bidx, bstar)
            above_sel = jnp.where(hit, abv, above_sel)
            rt = rt + jnp.sum(h)

        P = P | (bstar.astype(jnp.uint32) << jnp.uint32(shift))
        c_above = c_above + above_sel

        if r == 0:
            bs_u = bstar.astype(jnp.uint32)

            def cbody(i, off):
                bk = kv[pl.ds(i * 16, 16)]
                m = (bk >> jnp.uint32(shift)) == bs_u
                plsc.store_compressed(cbuf.at[pl.ds(off, 16)], bk, mask=m)
                return off + jnp.sum(jnp.where(m, 1, 0))

            m_local = lax.fori_loop(0, _NV, cbody, jnp.int32(0), unroll=False)
            compacted = True

    acc = jnp.zeros((16,), jnp.float32)

    def fbody(i, a):
        bk = kv[pl.ds(i * 16, 16)]
        sel = bk > P
        u = jnp.where(bk >= jnp.uint32(0x80000000),
                      bk - jnp.uint32(0x80000000), ~bk)
        z = plsc.bitcast(u, jnp.float32)
        return a + jnp.where(sel, _softplus16(z), 0.0)

    acc = lax.fori_loop(0, _NV, fbody, acc, unroll=8)
    local_sum = jnp.sum(acc)

    mf = n17.astype(jnp.float32)
    mlane = ((iota >> 2) & 1) == 1
    row = jnp.where(mlane, mf, jnp.full((16,), local_sum, jnp.float32))
    stage[pl.ds(0, 16)] = plsc.bitcast(row, jnp.int32)
    pltpu.sync_copy(stage.at[pl.ds(0, 16)], shf.at[pl.ds(wid * 16, 16)])

    @pl.when(wid == 0)
    def _():
        def fread(_unused):
            pltpu.sync_copy(shf, gathf)
            ok = jnp.zeros((16,), jnp.int32)
            for row_i in range(16):
                v = plsc.bitcast(gathf[pl.ds(row_i * 16, 16)], jnp.float32)
                ok = ok + jnp.where(mlane & (v == mf), 1, 0)
            return jnp.sum(ok)

        okf = fread(0)
        lax.while_loop(
            lambda cs: (cs[0] != jnp.int32(128)) & (cs[1] < _SPIN_LIM),
            lambda cs: (fread(0), cs[1] + 1),
            (okf, jnp.int32(0)))

        tot = jnp.zeros((16,), jnp.float32)
        for row_i in range(16):
            tot = tot + plsc.bitcast(gathf[pl.ds(row_i * 16, 16)], jnp.float32)
        ssum = jnp.sum(jnp.where(mlane, 0.0, tot)) * 0.125

        Pv = jnp.full((16,), P, jnp.uint32)
        u = jnp.where(Pv >= jnp.uint32(0x80000000),
                      Pv - jnp.uint32(0x80000000), ~Pv)
        lthr = _softplus16(plsc.bitcast(u, jnp.float32))
        kf = jnp.float32(_K)
        res = (ssum + (kf - c_above.astype(jnp.float32)) * lthr) * jnp.float32(1.0 / _K)
        stage[pl.ds(0, 16)] = plsc.bitcast(res, jnp.int32)
        pltpu.sync_copy(stage.at[pl.ds(0, 16)], out_hbm)


def kernel(inputs, targets):
    x = inputs.reshape(_N)
    t = targets.astype(jnp.int32).reshape(_N)
    s1 = jnp.sum(lax.bitcast_convert_type(x, jnp.int32), dtype=jnp.int32)
    s2 = jnp.sum(t, dtype=jnp.int32)
    nonce = s1 ^ (s2 * jnp.int32(-1640531527))
    nonce_arr = jnp.full((16,), nonce, jnp.int32)

    mesh = plsc.VectorSubcoreMesh(core_axis_name="c", subcore_axis_name="s",
                                  num_cores=1)
    run = pl.kernel(
        _sc_body,
        out_type=jax.ShapeDtypeStruct((16,), jnp.int32),
        mesh=mesh,
        scratch_types=[
            pltpu.VMEM((_CH,), jnp.float32),
            pltpu.VMEM((_CH,), jnp.int32),
            pltpu.VMEM((_CH,), jnp.uint32),
            pltpu.VMEM((_CH + 16,), jnp.uint32),
            pltpu.VMEM((128 * 16,), jnp.int32),
            pltpu.VMEM((128,), jnp.int32),
            pltpu.VMEM((2048,), jnp.int32),
            pltpu.VMEM((512,), jnp.int32),
            pltpu.VMEM((16,), jnp.int32),
            pltpu.VMEM_SHARED((2048,), jnp.int32),
            pltpu.VMEM_SHARED((2560,), jnp.int32),
            pltpu.VMEM_SHARED((256,), jnp.int32),
            pltpu.VMEM((256,), jnp.int32),
        ],
        compiler_params=_CP,
    )
    out = run(x, t, nonce_arr)
    return lax.bitcast_convert_type(out, jnp.float32)[0]


# speedup vs baseline: 1.0249x; 1.0249x over previous
import jax
import jax.numpy as jnp
from jax import lax
from jax.experimental import pallas as pl
from jax.experimental.pallas import tpu as pltpu
from jax.experimental.pallas import tpu_sc as plsc

_N = 16384
_K = 4915
_NW = 16
_CH = _N // _NW
_NV = _CH // 16

_ROUNDS = [(25, 7), (20, 5), (15, 5), (10, 5), (5, 5), (0, 5)]

_CP = pltpu.CompilerParams(needs_layout_passes=False)

_SPIN_LIM = jnp.int32(1 << 26)

_LOG1P = [5.237010936021136e-09, 0.9999989107138262, -0.4999622466391895,
          0.3328184407594258, -0.24635666572645035, 0.18468861839607467,
          -0.12526679277946434, 0.06651261739862285, -0.023038336454007536,
          0.0037526334774422208]


def _iota16():
    return lax.broadcasted_iota(jnp.int32, (16,), 0)


def _softplus16(z):
    w = jnp.exp(-jnp.abs(z))
    p = jnp.full(z.shape, jnp.float32(_LOG1P[-1]))
    for c in reversed(_LOG1P[:-1]):
        p = p * w + jnp.float32(c)
    return jnp.maximum(z, 0.0) + p


def _sc_body(x_hbm, t_hbm, nonce_hbm, out_hbm,
             xv, tv, kv, cbuf, hist, stage, gath0, gath, nv_vm,
             sh0, sh, shf, gathf):
    wid = lax.axis_index("s")
    base = wid * _CH

    pltpu.sync_copy(x_hbm.at[pl.ds(base, _CH)], xv)
    pltpu.sync_copy(t_hbm.at[pl.ds(base, _CH)], tv)
    pltpu.sync_copy(nonce_hbm, nv_vm)

    nonce = nv_vm[...]
    n17 = nonce & jnp.int32(0x1FFFF)
    nsh = n17 << jnp.int32(15)
    iota = _iota16()

    def prep(i, _):
        x = xv[pl.ds(i * 16, 16)]
        t = tv[pl.ds(i * 16, 16)].astype(jnp.float32)
        z = x * (1.0 - 2.0 * t)
        u = plsc.bitcast(z, jnp.uint32)
        neg = (u >> jnp.uint32(31)) == jnp.uint32(1)
        bk = jnp.where(neg, ~u, u | jnp.uint32(0x80000000))
        kv[pl.ds(i * 16, 16)] = bk
        return 0

    lax.fori_loop(0, _NV, prep, 0, unroll=False)

    P = jnp.uint32(0)
    c_above = jnp.int32(0)
    m_local = jnp.int32(_CH)
    compacted = False

    for r, (shift, width) in enumerate(_ROUNDS):
        nb = 1 << width
        nbv = nb // 16
        zero = jnp.zeros((16,), jnp.int32)

        def zbody(i, _):
            hist[pl.ds(i * 16, 16)] = zero
            return 0

        lax.fori_loop(0, nb, zbody, 0, unroll=False)

        ones = jnp.ones((16,), jnp.int32)
        lane_off = iota * nb
        mask_u = jnp.uint32((1 << width) - 1)

        if not compacted:
            def hbody(i, _):
                bk = kv[pl.ds(i * 16, 16)]
                b = ((bk >> jnp.uint32(shift)) & mask_u).astype(jnp.int32)
                plsc.addupdate_scatter(hist, [lane_off + b], ones)
                return 0

            lax.fori_loop(0, _NV, hbody, 0, unroll=False)
        else:
            pmask = jnp.uint32((~((1 << (shift + width)) - 1)) & 0xFFFFFFFF)
            Pcur = P

            def hbody(i, _):
                bk = cbuf[pl.ds(i * 16, 16)]
                inb = ((i * 16 + iota) < m_local) & ((bk & pmask) == Pcur)
                b = ((bk >> jnp.uint32(shift)) & mask_u).astype(jnp.int32)
                plsc.addupdate_scatter(hist, [lane_off + b], ones, mask=inb)
                return 0

            nvc = (m_local + jnp.int32(15)) >> jnp.int32(4)
            lax.fori_loop(0, nvc, hbody, 0, unroll=False)

        for j in range(nbv):
            acc = jnp.zeros((16,), jnp.int32)
            for l in range(16):
                acc = acc + hist[pl.ds(l * nb + j * 16, 16)]
            stage[pl.ds(j * 16, 16)] = acc | nsh

        if r == 0:
            pltpu.sync_copy(stage.at[pl.ds(0, nb)], sh0.at[pl.ds(wid * 128, 128)])
        else:
            pltpu.sync_copy(stage.at[pl.ds(0, nb)],
                            sh.at[pl.ds(((r - 1) * 16 + wid) * 32, 32)])

        g = gath0 if r == 0 else gath

        def read_and_check(_unused):
            if r == 0:
                pltpu.sync_copy(sh0, g)
            else:
                pltpu.sync_copy(sh.at[pl.ds((r - 1) * 512, 512)], g)
            okacc = jnp.zeros((16,), jnp.int32)
            for row in range(16):
                for j in range(nbv):
                    w = g[pl.ds(row * nb + j * 16, 16)]
                    okacc = okacc + jnp.where(
                        lax.shift_right_logical(w, jnp.int32(15)) == n17, 1, 0)
            return jnp.sum(okacc)

        okc = read_and_check(0)
        okc, _sp = lax.while_loop(
            lambda cs: (cs[0] != jnp.int32(16 * nb)) & (cs[1] < _SPIN_LIM),
            lambda cs: (read_and_check(0), cs[1] + 1),
            (okc, jnp.int32(0)))

        hv = []
        for j in range(nbv):
            acc = jnp.zeros((16,), jnp.int32)
            for row in range(16):
                acc = acc + (g[pl.ds(row * nb + j * 16, 16)] & jnp.int32(0x7FFF))
            hv.append(acc)

        need = jnp.int32(_K) - c_above
        rt = jnp.int32(0)
        bstar = jnp.int32(0)
        above_sel = jnp.int32(0)
        for j in reversed(range(nbv)):
            h = hv[j]
            rv = lax.rev(h, (0,))
            cs = plsc.cumsum(rv)
            a = lax.rev(cs, (0,)) + rt
            bmask = (a >= need) & ((a - h) < need)
            found = jnp.sum(jnp.where(bmask, 1, 0))
            bidx = jnp.sum(jnp.where(bmask, iota, 0))
            abv = jnp.sum(jnp.where(bmask, a - h, 0))
            hit = found > 0
            bstar = jnp.where(hit, j * 16 + bidx, bstar)
            above_sel = jnp.where(hit, abv, above_sel)
            rt = rt + jnp.sum(h)

        P = P | (bstar.astype(jnp.uint32) << jnp.uint32(shift))
        c_above = c_above + above_sel

        if r == 0:
            bs_u = bstar.astype(jnp.uint32)

            def cbody(i, off):
                bk = kv[pl.ds(i * 16, 16)]
                m = (bk >> jnp.uint32(shift)) == bs_u
                plsc.store_compressed(cbuf.at[pl.ds(off, 16)], bk, mask=m)
                return off + jnp.sum(jnp.where(m, 1, 0))

            m_local = lax.fori_loop(0, _NV, cbody, jnp.int32(0), unroll=False)
            compacted = True

    acc = jnp.zeros((16,), jnp.float32)

    def fbody(i, a):
        bk = kv[pl.ds(i * 16, 16)]
        sel = bk > P
        u = jnp.where(bk >= jnp.uint32(0x80000000),
                      bk - jnp.uint32(0x80000000), ~bk)
        z = plsc.bitcast(u, jnp.float32)
        return a + jnp.where(sel, _softplus16(z), 0.0)

    acc = lax.fori_loop(0, _NV, fbody, acc, unroll=False)
    local_sum = jnp.sum(acc)

    mf = n17.astype(jnp.float32)
    mlane = ((iota >> 2) & 1) == 1
    row = jnp.where(mlane, mf, jnp.full((16,), local_sum, jnp.float32))
    stage[pl.ds(0, 16)] = plsc.bitcast(row, jnp.int32)
    pltpu.sync_copy(stage.at[pl.ds(0, 16)], shf.at[pl.ds(wid * 16, 16)])

    @pl.when(wid == 0)
    def _():
        def fread(_unused):
            pltpu.sync_copy(shf, gathf)
            ok = jnp.zeros((16,), jnp.int32)
            for row_i in range(16):
                v = plsc.bitcast(gathf[pl.ds(row_i * 16, 16)], jnp.float32)
                ok = ok + jnp.where(mlane & (v == mf), 1, 0)
            return jnp.sum(ok)

        okf = fread(0)
        lax.while_loop(
            lambda cs: (cs[0] != jnp.int32(128)) & (cs[1] < _SPIN_LIM),
            lambda cs: (fread(0), cs[1] + 1),
            (okf, jnp.int32(0)))

        tot = jnp.zeros((16,), jnp.float32)
        for row_i in range(16):
            tot = tot + plsc.bitcast(gathf[pl.ds(row_i * 16, 16)], jnp.float32)
        ssum = jnp.sum(jnp.where(mlane, 0.0, tot)) * 0.125

        Pv = jnp.full((16,), P, jnp.uint32)
        u = jnp.where(Pv >= jnp.uint32(0x80000000),
                      Pv - jnp.uint32(0x80000000), ~Pv)
        lthr = _softplus16(plsc.bitcast(u, jnp.float32))
        kf = jnp.float32(_K)
        res = (ssum + (kf - c_above.astype(jnp.float32)) * lthr) * jnp.float32(1.0 / _K)
        stage[pl.ds(0, 16)] = plsc.bitcast(res, jnp.int32)
        pltpu.sync_copy(stage.at[pl.ds(0, 16)], out_hbm)


def kernel(inputs, targets):
    x = inputs.reshape(_N)
    t = targets.astype(jnp.int32).reshape(_N)
    s1 = jnp.sum(lax.bitcast_convert_type(x, jnp.int32), dtype=jnp.int32)
    s2 = jnp.sum(t, dtype=jnp.int32)
    nonce = s1 ^ (s2 * jnp.int32(-1640531527))
    nonce_arr = jnp.full((16,), nonce, jnp.int32)

    mesh = plsc.VectorSubcoreMesh(core_axis_name="c", subcore_axis_name="s",
                                  num_cores=1)
    run = pl.kernel(
        _sc_body,
        out_type=jax.ShapeDtypeStruct((16,), jnp.int32),
        mesh=mesh,
        scratch_types=[
            pltpu.VMEM((_CH,), jnp.float32),
            pltpu.VMEM((_CH,), jnp.int32),
            pltpu.VMEM((_CH,), jnp.uint32),
            pltpu.VMEM((_CH + 16,), jnp.uint32),
            pltpu.VMEM((128 * 16,), jnp.int32),
            pltpu.VMEM((128,), jnp.int32),
            pltpu.VMEM((2048,), jnp.int32),
            pltpu.VMEM((512,), jnp.int32),
            pltpu.VMEM((16,), jnp.int32),
            pltpu.VMEM_SHARED((2048,), jnp.int32),
            pltpu.VMEM_SHARED((2560,), jnp.int32),
            pltpu.VMEM_SHARED((256,), jnp.int32),
            pltpu.VMEM((256,), jnp.int32),
        ],
        compiler_params=_CP,
    )
    out = run(x, t, nonce_arr)
    return lax.bitcast_convert_type(out, jnp.float32)[0]
